# baseline (speedup 1.0000x reference)
import functools
import math

import jax
import jax.numpy as jnp
from jax.experimental import pallas as pl
from jax.experimental.pallas import tpu as pltpu

_SCALE = 3.0
_SHIFTS = 16
_BLOCK = 1024
_C = 2.0 * _SCALE * _SCALE * math.log2(math.e)
_LN2 = math.log(2.0)


def _unit_rows(x):
    n = jnp.sqrt(jnp.sum(x * x, axis=-1, keepdims=True))
    return x / jnp.maximum(n, 1e-12)


def _nca_body(xa_ref, p_ref, ta_ref, tb_ref, t2b_ref, out_ref,
              pn_ref, ones_ref, g_ref, virt_ref, xr_ref, sem, acc_ref,
              *, nblk, block, ncls):
    s = pl.program_id(0)
    cp = pltpu.make_async_copy(xa_ref, xr_ref.at[jax.lax.rem(s, 2)], sem)

    @pl.when(s < nblk)
    def _carry():
        cp.start()

    @pl.when(s == 0)
    def _init():
        acc_ref[0] = 0.0
        acc_ref[1] = 0.0
        pn_ref[:, :] = _unit_rows(p_ref[:, :]).astype(jnp.bfloat16)
        ones_ref[:, :] = jnp.ones_like(ones_ref)

    cols = jax.lax.broadcasted_iota(jnp.int32, (block, ncls), 1)

    def _rowsum(a):
        return jax.lax.dot_general(
            a.astype(jnp.bfloat16), ones_ref[:, :], (((1,), (0,)), ((), ())),
            preferred_element_type=jnp.float32)[:, :1]

    def _y_lse(x, sqsum):
        inv = _C / jnp.maximum(jnp.sqrt(sqsum), 1e-12)
        y = jax.lax.dot_general(
            (x * inv).astype(jnp.bfloat16), pn_ref[:, :],
            (((1,), (1,)), ((), ())), preferred_element_type=jnp.float32)
        lse = jnp.log2(_rowsum(jnp.exp2(y)))
        return y, lse

    @pl.when(s < nblk)
    def _pass1():
        x = xa_ref[:, :]
        y, lse = _y_lse(x, _rowsum(x * x))
        lt = _rowsum(jnp.where(cols == ta_ref[0, :, :], y, 0.0))
        acc_ref[0] += jnp.sum(lse - lt)
        gval = jnp.clip(lt * (1.0 / _C), 0.0, 1.0)
        g_ref[jax.lax.rem(s, 2)] = gval

        @pl.when(s == 0)
        def _pin():
            g_ref[2] = gval

    @pl.when(s > 0)
    def _pass2():
        gb = g_ref[jax.lax.rem(s - 1, 2)]
        gb1 = g_ref[jnp.where(s < nblk, jax.lax.rem(s, 2), 2)]
        g2 = jnp.concatenate([gb[_SHIFTS:, :], gb1[:_SHIFTS, :]], axis=0)
        lam = jnp.clip((gb + 1.0 - g2) * 0.5, 0.0, 1.0)
        xb = xr_ref[jax.lax.rem(s - 1, 2)]
        hi = block - _SHIFTS
        lam_h = lam[:hi, :]
        virt_ref[:hi, :] = (lam_h * xb[:hi, :] +
                            (1.0 - lam_h) * xb[_SHIFTS:, :])
        lam_t = lam[hi:, :]
        virt_ref[hi:, :] = (lam_t * xb[hi:, :] +
                            (1.0 - lam_t) * xa_ref[:_SHIFTS, :])
        virt = virt_ref[:, :]
        y, lse = _y_lse(virt, _rowsum(virt * virt))
        l1 = _rowsum(jnp.where(cols == tb_ref[0, :, :], y, 0.0))
        l2 = _rowsum(jnp.where(cols == t2b_ref[0, :, :], y, 0.0))
        acc_ref[1] += jnp.sum(lse - lam * l1 - (1.0 - lam) * l2)

    @pl.when(s < nblk)
    def _carry_done():
        cp.wait()

    @pl.when(s == nblk)
    def _fin():
        out_ref[:, :] = jnp.full(
            (1, 1), _LN2 * (acc_ref[0] + acc_ref[1]) / (nblk * block),
            jnp.float32)


@functools.partial(jax.jit, static_argnames=("interpret",))
def kernel(X, T, proxies, interpret=False):
    n, e = X.shape
    ncls = proxies.shape[0]
    block = _BLOCK
    nblk = n // block

    T = T.astype(jnp.int32)
    t_col = T.reshape(nblk, block, 1)
    t2_col = jnp.roll(T, -_SHIFTS).reshape(nblk, block, 1)

    out = pl.pallas_call(
        functools.partial(_nca_body, nblk=nblk, block=block, ncls=ncls),
        grid=(nblk + 1,),
        in_specs=[
            pl.BlockSpec((block, e), lambda s: (jax.lax.rem(s, nblk), 0)),
            pl.BlockSpec((ncls, e), lambda s: (0, 0)),
            pl.BlockSpec((1, block, 1),
                         lambda s: (jax.lax.rem(s, nblk), 0, 0)),
            pl.BlockSpec((1, block, 1),
                         lambda s: (jnp.maximum(s - 1, 0), 0, 0)),
            pl.BlockSpec((1, block, 1),
                         lambda s: (jnp.maximum(s - 1, 0), 0, 0)),
        ],
        out_specs=pl.BlockSpec((1, 1), lambda s: (0, 0)),
        out_shape=jax.ShapeDtypeStruct((1, 1), jnp.float32),
        scratch_shapes=[
            pltpu.VMEM((ncls, e), jnp.bfloat16),
            pltpu.VMEM((e, 128), jnp.bfloat16),
            pltpu.VMEM((3, block, 1), jnp.float32),
            pltpu.VMEM((block, e), jnp.float32),
            pltpu.VMEM((2, block, e), jnp.float32),
            pltpu.SemaphoreType.DMA,
            pltpu.SMEM((2,), jnp.float32),
        ],
        interpret=interpret,
    )(X, proxies, t_col, t_col, t2_col)
    return out[0, 0]


# speedup vs baseline: 1.1286x; 1.1286x over previous
import functools
import math

import jax
import jax.numpy as jnp
from jax.experimental import pallas as pl
from jax.experimental.pallas import tpu as pltpu

_SCALE = 3.0
_SHIFTS = 16
_BLOCK = 2048
_C = 2.0 * _SCALE * _SCALE * math.log2(math.e)
_LN2 = math.log(2.0)


def _unit_rows(x):
    n = jnp.sqrt(jnp.sum(x * x, axis=-1, keepdims=True))
    return x / jnp.maximum(n, 1e-12)


def _nca_body(xa_ref, p_ref, ta_ref, tb_ref, t2b_ref, out_ref,
              pn_ref, ones_ref, g_ref, xr_ref, acc_ref,
              *, nblk, block, ncls):
    s = pl.program_id(0)

    @pl.when(s == 0)
    def _init():
        acc_ref[0] = 0.0
        acc_ref[1] = 0.0
        pn_ref[:, :] = _unit_rows(p_ref[:, :]).astype(jnp.bfloat16)
        ones_ref[:, :] = jnp.ones_like(ones_ref)

    cols = jax.lax.broadcasted_iota(jnp.int32, (block, ncls), 1)
    bzero = jnp.bfloat16(0.0)

    def _rowsum(a16):
        return jax.lax.dot_general(
            a16, ones_ref[:, :], (((1,), (0,)), ((), ())),
            preferred_element_type=jnp.float32)[:, :1]

    def _y_lse(x16):
        inv = _C / jnp.maximum(jnp.sqrt(_rowsum(x16 * x16)), 1e-12)
        m = jax.lax.dot_general(
            x16, pn_ref[:, :], (((1,), (1,)), ((), ())),
            preferred_element_type=jnp.float32)
        y16 = (m * inv).astype(jnp.bfloat16)
        lse = jnp.log2(_rowsum(jnp.exp2(y16)))
        return y16, lse

    @pl.when(s < nblk)
    def _pass1():
        x16 = xa_ref[:, :].astype(jnp.bfloat16)
        xr_ref[jax.lax.rem(s, 2)] = x16
        y16, lse = _y_lse(x16)
        lt = _rowsum(jnp.where(cols == ta_ref[0, :, :], y16, bzero))
        acc_ref[0] += jnp.sum(lse - lt)
        gval = jnp.clip(lt * (1.0 / _C), 0.0, 1.0)
        g_ref[jax.lax.rem(s, 2)] = gval

        @pl.when(s == 0)
        def _pin():
            g_ref[2] = gval

    @pl.when(s > 0)
    def _pass2():
        gb = g_ref[jax.lax.rem(s - 1, 2)]
        gb1 = g_ref[jnp.where(s < nblk, jax.lax.rem(s, 2), 2)]
        g2 = jnp.concatenate([gb[_SHIFTS:, :], gb1[:_SHIFTS, :]], axis=0)
        lam = jnp.clip((gb + 1.0 - g2) * 0.5, 0.0, 1.0)
        xb16 = xr_ref[jax.lax.rem(s - 1, 2)]
        xs16 = jnp.concatenate(
            [xb16[_SHIFTS:, :], xa_ref[:_SHIFTS, :].astype(jnp.bfloat16)],
            axis=0)
        lam16 = lam.astype(jnp.bfloat16)
        v16 = lam16 * xb16 + (jnp.bfloat16(1.0) - lam16) * xs16
        y16, lse = _y_lse(v16)
        l1 = _rowsum(jnp.where(cols == tb_ref[0, :, :], y16, bzero))
        l2 = _rowsum(jnp.where(cols == t2b_ref[0, :, :], y16, bzero))
        acc_ref[1] += jnp.sum(lse - lam * l1 - (1.0 - lam) * l2)

    @pl.when(s == nblk)
    def _fin():
        out_ref[:, :] = jnp.full(
            (1, 1), _LN2 * (acc_ref[0] + acc_ref[1]) / (nblk * block),
            jnp.float32)


@functools.partial(jax.jit, static_argnames=("interpret",))
def kernel(X, T, proxies, interpret=False):
    n, e = X.shape
    ncls = proxies.shape[0]
    block = _BLOCK
    nblk = n // block

    T = T.astype(jnp.int32)
    t_col = T.reshape(nblk, block, 1)
    t2_col = jnp.roll(T, -_SHIFTS).reshape(nblk, block, 1)

    out = pl.pallas_call(
        functools.partial(_nca_body, nblk=nblk, block=block, ncls=ncls),
        grid=(nblk + 1,),
        in_specs=[
            pl.BlockSpec((block, e), lambda s: (jax.lax.rem(s, nblk), 0)),
            pl.BlockSpec((ncls, e), lambda s: (0, 0)),
            pl.BlockSpec((1, block, 1),
                         lambda s: (jax.lax.rem(s, nblk), 0, 0)),
            pl.BlockSpec((1, block, 1),
                         lambda s: (jnp.maximum(s - 1, 0), 0, 0)),
            pl.BlockSpec((1, block, 1),
                         lambda s: (jnp.maximum(s - 1, 0), 0, 0)),
        ],
        out_specs=pl.BlockSpec((1, 1), lambda s: (0, 0)),
        out_shape=jax.ShapeDtypeStruct((1, 1), jnp.float32),
        scratch_shapes=[
            pltpu.VMEM((ncls, e), jnp.bfloat16),
            pltpu.VMEM((e, 128), jnp.bfloat16),
            pltpu.VMEM((3, block, 1), jnp.float32),
            pltpu.VMEM((2, block, e), jnp.bfloat16),
            pltpu.SMEM((2,), jnp.float32),
        ],
        interpret=interpret,
    )(X, proxies, t_col, t_col, t2_col)
    return out[0, 0]


# speedup vs baseline: 1.1629x; 1.0304x over previous
import functools
import math

import jax
import jax.numpy as jnp
from jax.experimental import pallas as pl
from jax.experimental.pallas import tpu as pltpu

_SCALE = 3.0
_SHIFTS = 16
_BLOCK = 2048
_C = 2.0 * _SCALE * _SCALE * math.log2(math.e)
_LN2 = math.log(2.0)


def _unit_rows(x):
    n = jnp.sqrt(jnp.sum(x * x, axis=-1, keepdims=True))
    return x / jnp.maximum(n, 1e-12)


def _nca_body(xa_ref, p_ref, ta_ref, tb_ref, t2b_ref, out_ref,
              pn_ref, ones_ref, g_ref, g0h_ref, xr_ref, x0h_ref, acc_ref,
              *, nblk, block, ncls):
    s = pl.program_id(0)

    @pl.when(s == 0)
    def _init():
        acc_ref[0] = 0.0
        acc_ref[1] = 0.0
        pn_ref[:, :] = _unit_rows(p_ref[:, :]).astype(jnp.bfloat16)
        ones_ref[:, :] = jnp.ones_like(ones_ref)

    cols = jax.lax.broadcasted_iota(jnp.int16, (block, ncls), 1)
    bzero = jnp.bfloat16(0.0)

    def _rowsum(a16):
        return jax.lax.dot_general(
            a16, ones_ref[:, :], (((1,), (0,)), ((), ())),
            preferred_element_type=jnp.float32)[:, :1]

    def _y_lse(x16):
        sq = jnp.sum(x16 * x16, axis=1, keepdims=True, dtype=jnp.float32)
        inv = _C / jnp.maximum(jnp.sqrt(sq), 1e-12)
        m = jax.lax.dot_general(
            x16, pn_ref[:, :], (((1,), (1,)), ((), ())),
            preferred_element_type=jnp.float32)
        y16 = (m * inv).astype(jnp.bfloat16)
        lse = jnp.log2(_rowsum(jnp.exp2(y16)))
        return y16, lse

    @pl.when(s < nblk)
    def _pass1():
        x16 = xa_ref[:, :].astype(jnp.bfloat16)
        xr_ref[jax.lax.rem(s, 3)] = x16
        y16, lse = _y_lse(x16)
        lt = _rowsum(jnp.where(cols == ta_ref[0, :, :].astype(jnp.int16),
                               y16, bzero))
        acc_ref[0] += jnp.sum(lse - lt)
        gval = jnp.clip(lt * (1.0 / _C), 0.0, 1.0)
        g_ref[jax.lax.rem(s, 3)] = gval

        @pl.when(s == 0)
        def _pin():
            g0h_ref[:, :] = gval[:_SHIFTS, :]
            x0h_ref[:, :] = x16[:_SHIFTS, :]

    @pl.when(s >= 2)
    def _pass2():
        in_ring = (s - 1) < nblk
        gb = g_ref[jax.lax.rem(s - 2, 3)]
        gh = jnp.where(in_ring, g_ref[jax.lax.rem(s - 1, 3)][:_SHIFTS, :],
                       g0h_ref[:, :])
        g2 = jnp.concatenate([gb[_SHIFTS:, :], gh], axis=0)
        lam = jnp.clip((gb + 1.0 - g2) * 0.5, 0.0, 1.0)
        xb16 = xr_ref[jax.lax.rem(s - 2, 3)]
        xh = jnp.where(in_ring, xr_ref[jax.lax.rem(s - 1, 3)][:_SHIFTS, :],
                       x0h_ref[:, :])
        xs16 = jnp.concatenate([xb16[_SHIFTS:, :], xh], axis=0)
        lam16 = lam.astype(jnp.bfloat16)
        v16 = lam16 * xb16 + (jnp.bfloat16(1.0) - lam16) * xs16
        y16, lse = _y_lse(v16)
        l1 = _rowsum(jnp.where(cols == tb_ref[0, :, :].astype(jnp.int16),
                               y16, bzero))
        l2 = _rowsum(jnp.where(cols == t2b_ref[0, :, :].astype(jnp.int16),
                               y16, bzero))
        acc_ref[1] += jnp.sum(lse - lam * l1 - (1.0 - lam) * l2)

    @pl.when(s == nblk + 1)
    def _fin():
        out_ref[:, :] = jnp.full(
            (1, 1), _LN2 * (acc_ref[0] + acc_ref[1]) / (nblk * block),
            jnp.float32)


@functools.partial(jax.jit, static_argnames=("interpret",))
def kernel(X, T, proxies, interpret=False):
    n, e = X.shape
    ncls = proxies.shape[0]
    block = _BLOCK
    nblk = n // block

    T = T.astype(jnp.int32)
    t_col = T.reshape(nblk, block, 1)
    t2_col = jnp.roll(T, -_SHIFTS).reshape(nblk, block, 1)

    out = pl.pallas_call(
        functools.partial(_nca_body, nblk=nblk, block=block, ncls=ncls),
        grid=(nblk + 2,),
        in_specs=[
            pl.BlockSpec((block, e), lambda s: (jnp.minimum(s, nblk - 1), 0)),
            pl.BlockSpec((ncls, e), lambda s: (0, 0)),
            pl.BlockSpec((1, block, 1),
                         lambda s: (jnp.minimum(s, nblk - 1), 0, 0)),
            pl.BlockSpec((1, block, 1),
                         lambda s: (jnp.maximum(s - 2, 0), 0, 0)),
            pl.BlockSpec((1, block, 1),
                         lambda s: (jnp.maximum(s - 2, 0), 0, 0)),
        ],
        out_specs=pl.BlockSpec((1, 1), lambda s: (0, 0)),
        out_shape=jax.ShapeDtypeStruct((1, 1), jnp.float32),
        scratch_shapes=[
            pltpu.VMEM((ncls, e), jnp.bfloat16),
            pltpu.VMEM((e, 128), jnp.bfloat16),
            pltpu.VMEM((3, block, 1), jnp.float32),
            pltpu.VMEM((_SHIFTS, 1), jnp.float32),
            pltpu.VMEM((3, block, e), jnp.bfloat16),
            pltpu.VMEM((_SHIFTS, e), jnp.bfloat16),
            pltpu.SMEM((2,), jnp.float32),
        ],
        interpret=interpret,
    )(X, proxies, t_col, t_col, t2_col)
    return out[0, 0]


# speedup vs baseline: 1.2343x; 1.0614x over previous
import functools
import math

import jax
import jax.numpy as jnp
from jax.experimental import pallas as pl
from jax.experimental.pallas import tpu as pltpu

_SCALE = 3.0
_SHIFTS = 16
_BLOCK = 2048
_C = 2.0 * _SCALE * _SCALE * math.log2(math.e)
_LN2 = math.log(2.0)


def _unit_rows(x):
    n = jnp.sqrt(jnp.sum(x * x, axis=-1, keepdims=True))
    return x / jnp.maximum(n, 1e-12)


def _nca_body(xa_ref, p_ref, ta_ref, tb_ref, t2b_ref, out_ref,
              pn_ref, ones_ref, g_ref, g0h_ref, n_ref, n0h_ref, cr_ref,
              yr_ref, y0h_ref, xt_ref, x0h_ref, acc_ref,
              *, nblk, block, ncls):
    s = pl.program_id(0)
    hi = block - _SHIFTS

    @pl.when(s == 0)
    def _init():
        acc_ref[0] = 0.0
        acc_ref[1] = 0.0
        pn_ref[:, :] = _unit_rows(p_ref[:, :]).astype(jnp.bfloat16)
        ones_ref[:, :] = jnp.ones_like(ones_ref)

    cols = jax.lax.broadcasted_iota(jnp.int16, (block, ncls), 1)
    bzero = jnp.bfloat16(0.0)

    def _rowsum(a16):
        return jax.lax.dot_general(
            a16, ones_ref[:, :], (((1,), (0,)), ((), ())),
            preferred_element_type=jnp.float32)[:, :1]

    @pl.when(jnp.logical_and(s >= 1, s <= nblk))
    def _cross_fix():
        xh16 = jnp.where(s < nblk, xa_ref[:_SHIFTS, :].astype(jnp.bfloat16),
                         x0h_ref[:, :])
        cr_ref[jax.lax.rem(s - 1, 3), hi:, :] = jnp.sum(
            xt_ref[:, :] * xh16, axis=1, keepdims=True, dtype=jnp.float32)

    @pl.when(s < nblk)
    def _pass1():
        x16 = xa_ref[:, :].astype(jnp.bfloat16)
        sq = jnp.sum(x16 * x16, axis=1, keepdims=True, dtype=jnp.float32)
        nb = jnp.sqrt(sq)
        inv = _C / jnp.maximum(nb, 1e-12)
        m = jax.lax.dot_general(
            x16, pn_ref[:, :], (((1,), (1,)), ((), ())),
            preferred_element_type=jnp.float32)
        y16 = (m * inv).astype(jnp.bfloat16)
        yr_ref[jax.lax.rem(s, 3)] = y16
        lse = jnp.log2(_rowsum(jnp.exp2(y16)))
        lt = _rowsum(jnp.where(cols == ta_ref[0, :, :].astype(jnp.int16),
                               y16, bzero))
        acc_ref[0] += jnp.sum(lse - lt)
        gval = jnp.clip(lt * (1.0 / _C), 0.0, 1.0)
        g_ref[jax.lax.rem(s, 3)] = gval
        n_ref[jax.lax.rem(s, 3)] = nb
        cr_ref[jax.lax.rem(s, 3), :hi, :] = jnp.sum(
            x16[:hi, :] * x16[_SHIFTS:, :], axis=1, keepdims=True,
            dtype=jnp.float32)
        xt_ref[:, :] = x16[hi:, :]

        @pl.when(s == 0)
        def _pin():
            g0h_ref[:, :] = gval[:_SHIFTS, :]
            n0h_ref[:, :] = nb[:_SHIFTS, :]
            y0h_ref[:, :] = y16[:_SHIFTS, :]
            x0h_ref[:, :] = x16[:_SHIFTS, :]

    @pl.when(s >= 2)
    def _pass2():
        in_ring = (s - 1) < nblk
        gb = g_ref[jax.lax.rem(s - 2, 3)]
        gh = jnp.where(in_ring, g_ref[jax.lax.rem(s - 1, 3)][:_SHIFTS, :],
                       g0h_ref[:, :])
        g2 = jnp.concatenate([gb[_SHIFTS:, :], gh], axis=0)
        lam = jnp.clip((gb + 1.0 - g2) * 0.5, 0.0, 1.0)
        oml = 1.0 - lam
        nb = n_ref[jax.lax.rem(s - 2, 3)]
        nh = jnp.where(in_ring, n_ref[jax.lax.rem(s - 1, 3)][:_SHIFTS, :],
                       n0h_ref[:, :])
        ns = jnp.concatenate([nb[_SHIFTS:, :], nh], axis=0)
        cross = cr_ref[jax.lax.rem(s - 2, 3)]
        vn2 = (lam * lam * nb * nb + oml * oml * ns * ns +
               2.0 * (lam * oml) * cross)
        vmax = jnp.maximum(jnp.sqrt(jnp.maximum(vn2, 0.0)), 1e-12)
        alpha = (lam * nb / vmax).astype(jnp.bfloat16)
        beta = (oml * ns / vmax).astype(jnp.bfloat16)
        yb16 = yr_ref[jax.lax.rem(s - 2, 3)]
        yh16 = jnp.where(in_ring, yr_ref[jax.lax.rem(s - 1, 3)][:_SHIFTS, :],
                         y0h_ref[:, :])
        ys16 = jnp.concatenate([yb16[_SHIFTS:, :], yh16], axis=0)
        yv16 = alpha * yb16 + beta * ys16
        lse = jnp.log2(_rowsum(jnp.exp2(yv16)))
        l1 = _rowsum(jnp.where(cols == tb_ref[0, :, :].astype(jnp.int16),
                               yv16, bzero))
        l2 = _rowsum(jnp.where(cols == t2b_ref[0, :, :].astype(jnp.int16),
                               yv16, bzero))
        acc_ref[1] += jnp.sum(lse - lam * l1 - oml * l2)

    @pl.when(s == nblk + 1)
    def _fin():
        out_ref[:, :] = jnp.full(
            (1, 1), _LN2 * (acc_ref[0] + acc_ref[1]) / (nblk * block),
            jnp.float32)


@functools.partial(jax.jit, static_argnames=("interpret",))
def kernel(X, T, proxies, interpret=False):
    n, e = X.shape
    ncls = proxies.shape[0]
    block = _BLOCK
    nblk = n // block

    T = T.astype(jnp.int32)
    t_col = T.reshape(nblk, block, 1)
    t2_col = jnp.roll(T, -_SHIFTS).reshape(nblk, block, 1)

    out = pl.pallas_call(
        functools.partial(_nca_body, nblk=nblk, block=block, ncls=ncls),
        grid=(nblk + 2,),
        in_specs=[
            pl.BlockSpec((block, e), lambda s: (jnp.minimum(s, nblk - 1), 0)),
            pl.BlockSpec((ncls, e), lambda s: (0, 0)),
            pl.BlockSpec((1, block, 1),
                         lambda s: (jnp.minimum(s, nblk - 1), 0, 0)),
            pl.BlockSpec((1, block, 1),
                         lambda s: (jnp.maximum(s - 2, 0), 0, 0)),
            pl.BlockSpec((1, block, 1),
                         lambda s: (jnp.maximum(s - 2, 0), 0, 0)),
        ],
        out_specs=pl.BlockSpec((1, 1), lambda s: (0, 0)),
        out_shape=jax.ShapeDtypeStruct((1, 1), jnp.float32),
        scratch_shapes=[
            pltpu.VMEM((ncls, e), jnp.bfloat16),
            pltpu.VMEM((e, 128), jnp.bfloat16),
            pltpu.VMEM((3, block, 1), jnp.float32),
            pltpu.VMEM((_SHIFTS, 1), jnp.float32),
            pltpu.VMEM((3, block, 1), jnp.float32),
            pltpu.VMEM((_SHIFTS, 1), jnp.float32),
            pltpu.VMEM((3, block, 1), jnp.float32),
            pltpu.VMEM((3, block, ncls), jnp.bfloat16),
            pltpu.VMEM((_SHIFTS, ncls), jnp.bfloat16),
            pltpu.VMEM((_SHIFTS, e), jnp.bfloat16),
            pltpu.VMEM((_SHIFTS, e), jnp.bfloat16),
            pltpu.SMEM((2,), jnp.float32),
        ],
        interpret=interpret,
    )(X, proxies, t_col, t_col, t2_col)
    return out[0, 0]


# speedup vs baseline: 1.3300x; 1.0776x over previous
import functools
import math

import jax
import jax.numpy as jnp
from jax.experimental import pallas as pl
from jax.experimental.pallas import tpu as pltpu

_SCALE = 3.0
_SHIFTS = 16
_BLOCK = 2048
_C = 2.0 * _SCALE * _SCALE * math.log2(math.e)
_LN2 = math.log(2.0)


def _unit_rows(x):
    n = jnp.sqrt(jnp.sum(x * x, axis=-1, keepdims=True))
    return x / jnp.maximum(n, 1e-12)


def _nca_body(xa_ref, p_ref, ta_ref, tb_ref, t2b_ref, out_ref,
              pn_ref, ones_ref, g_ref, g0h_ref, n_ref, n0h_ref, cr_ref,
              yr_ref, y0h_ref, xt_ref, x0h_ref, acc_ref,
              *, nblk, block, ncls):
    s = pl.program_id(0)
    hi = block - _SHIFTS

    @pl.when(s == 0)
    def _init():
        acc_ref[0] = 0.0
        acc_ref[1] = 0.0
        pn_ref[:, :] = _unit_rows(p_ref[:, :]).astype(jnp.bfloat16)
        ones_ref[:, :] = jnp.ones_like(ones_ref)

    cols = jax.lax.broadcasted_iota(jnp.int16, (block, ncls), 1)
    bzero = jnp.bfloat16(0.0)

    def _rowsum(a16):
        return jax.lax.dot_general(
            a16, ones_ref[:, :], (((1,), (0,)), ((), ())),
            preferred_element_type=jnp.float32)[:, :1]

    @pl.when(jnp.logical_and(s >= 1, s <= nblk))
    def _cross_fix():
        xh16 = jnp.where(s < nblk, xa_ref[:_SHIFTS, :].astype(jnp.bfloat16),
                         x0h_ref[:, :])
        cr_ref[jax.lax.rem(s - 1, 3), hi:, :] = jnp.sum(
            xt_ref[:, :] * xh16, axis=1, keepdims=True, dtype=jnp.float32)

    @pl.when(s < nblk)
    def _pass1():
        x16 = xa_ref[:, :].astype(jnp.bfloat16)
        sq = jnp.sum(x16 * x16, axis=1, keepdims=True, dtype=jnp.float32)
        nb = jnp.sqrt(sq)
        inv = _C / jnp.maximum(nb, 1e-12)
        m = jax.lax.dot_general(
            x16, pn_ref[:, :], (((1,), (1,)), ((), ())),
            preferred_element_type=jnp.float32)
        y16 = (m * inv).astype(jnp.bfloat16)
        yr_ref[jax.lax.rem(s, 3)] = y16
        lse = jnp.log2(jnp.sum(jnp.exp2(y16), axis=1, keepdims=True,
                               dtype=jnp.float32))
        lt = _rowsum(jnp.where(cols == ta_ref[0, :, :].astype(jnp.int16),
                               y16, bzero))
        acc_ref[0] += jnp.sum(lse - lt)
        gval = jnp.clip(lt * (1.0 / _C), 0.0, 1.0)
        g_ref[jax.lax.rem(s, 3)] = gval
        n_ref[jax.lax.rem(s, 3)] = nb
        cr_ref[jax.lax.rem(s, 3), :hi, :] = jnp.sum(
            x16[:hi, :] * x16[_SHIFTS:, :], axis=1, keepdims=True,
            dtype=jnp.float32)
        xt_ref[:, :] = x16[hi:, :]

        @pl.when(s == 0)
        def _pin():
            g0h_ref[:, :] = gval[:_SHIFTS, :]
            n0h_ref[:, :] = nb[:_SHIFTS, :]
            y0h_ref[:, :] = y16[:_SHIFTS, :]
            x0h_ref[:, :] = x16[:_SHIFTS, :]

    @pl.when(s >= 2)
    def _pass2():
        in_ring = (s - 1) < nblk
        gb = g_ref[jax.lax.rem(s - 2, 3)]
        gh = jnp.where(in_ring, g_ref[jax.lax.rem(s - 1, 3)][:_SHIFTS, :],
                       g0h_ref[:, :])
        g2 = jnp.concatenate([gb[_SHIFTS:, :], gh], axis=0)
        lam = jnp.clip((gb + 1.0 - g2) * 0.5, 0.0, 1.0)
        oml = 1.0 - lam
        nb = n_ref[jax.lax.rem(s - 2, 3)]
        nh = jnp.where(in_ring, n_ref[jax.lax.rem(s - 1, 3)][:_SHIFTS, :],
                       n0h_ref[:, :])
        ns = jnp.concatenate([nb[_SHIFTS:, :], nh], axis=0)
        cross = cr_ref[jax.lax.rem(s - 2, 3)]
        vn2 = (lam * lam * nb * nb + oml * oml * ns * ns +
               2.0 * (lam * oml) * cross)
        vmax = jnp.maximum(jnp.sqrt(jnp.maximum(vn2, 0.0)), 1e-12)
        alpha = (lam * nb / vmax).astype(jnp.bfloat16)
        beta = (oml * ns / vmax).astype(jnp.bfloat16)
        yb16 = yr_ref[jax.lax.rem(s - 2, 3)]
        yh16 = jnp.where(in_ring, yr_ref[jax.lax.rem(s - 1, 3)][:_SHIFTS, :],
                         y0h_ref[:, :])
        ys16 = jnp.concatenate([yb16[_SHIFTS:, :], yh16], axis=0)
        yv16 = alpha * yb16 + beta * ys16
        lse = jnp.log2(jnp.sum(jnp.exp2(yv16), axis=1, keepdims=True,
                               dtype=jnp.float32))
        l1 = _rowsum(jnp.where(cols == tb_ref[0, :, :].astype(jnp.int16),
                               yv16, bzero))
        l2 = _rowsum(jnp.where(cols == t2b_ref[0, :, :].astype(jnp.int16),
                               yv16, bzero))
        acc_ref[1] += jnp.sum(lse - lam * l1 - oml * l2)

    @pl.when(s == nblk + 1)
    def _fin():
        out_ref[:, :] = jnp.full(
            (1, 1), _LN2 * (acc_ref[0] + acc_ref[1]) / (nblk * block),
            jnp.float32)


@functools.partial(jax.jit, static_argnames=("interpret",))
def kernel(X, T, proxies, interpret=False):
    n, e = X.shape
    ncls = proxies.shape[0]
    block = _BLOCK
    nblk = n // block

    T = T.astype(jnp.int32)
    t_col = T.reshape(nblk, block, 1)
    t2_col = jnp.roll(T, -_SHIFTS).reshape(nblk, block, 1)

    out = pl.pallas_call(
        functools.partial(_nca_body, nblk=nblk, block=block, ncls=ncls),
        grid=(nblk + 2,),
        in_specs=[
            pl.BlockSpec((block, e), lambda s: (jnp.minimum(s, nblk - 1), 0)),
            pl.BlockSpec((ncls, e), lambda s: (0, 0)),
            pl.BlockSpec((1, block, 1),
                         lambda s: (jnp.minimum(s, nblk - 1), 0, 0)),
            pl.BlockSpec((1, block, 1),
                         lambda s: (jnp.maximum(s - 2, 0), 0, 0)),
            pl.BlockSpec((1, block, 1),
                         lambda s: (jnp.maximum(s - 2, 0), 0, 0)),
        ],
        out_specs=pl.BlockSpec((1, 1), lambda s: (0, 0)),
        out_shape=jax.ShapeDtypeStruct((1, 1), jnp.float32),
        scratch_shapes=[
            pltpu.VMEM((ncls, e), jnp.bfloat16),
            pltpu.VMEM((e, 128), jnp.bfloat16),
            pltpu.VMEM((3, block, 1), jnp.float32),
            pltpu.VMEM((_SHIFTS, 1), jnp.float32),
            pltpu.VMEM((3, block, 1), jnp.float32),
            pltpu.VMEM((_SHIFTS, 1), jnp.float32),
            pltpu.VMEM((3, block, 1), jnp.float32),
            pltpu.VMEM((3, block, ncls), jnp.bfloat16),
            pltpu.VMEM((_SHIFTS, ncls), jnp.bfloat16),
            pltpu.VMEM((_SHIFTS, e), jnp.bfloat16),
            pltpu.VMEM((_SHIFTS, e), jnp.bfloat16),
            pltpu.SMEM((2,), jnp.float32),
        ],
        interpret=interpret,
    )(X, proxies, t_col, t_col, t2_col)
    return out[0, 0]


# speedup vs baseline: 1.3438x; 1.0103x over previous
import functools
import math

import jax
import jax.numpy as jnp
from jax.experimental import pallas as pl
from jax.experimental.pallas import tpu as pltpu

_SCALE = 3.0
_SHIFTS = 16
_BLOCK = 2048
_C = 2.0 * _SCALE * _SCALE * math.log2(math.e)
_LN2 = math.log(2.0)


def _unit_rows(x):
    n = jnp.sqrt(jnp.sum(x * x, axis=-1, keepdims=True))
    return x / jnp.maximum(n, 1e-12)


def _nca_body(xa_ref, p_ref, ta_ref, tb_ref, t2b_ref, out_ref,
              pn_ref, ones_ref, g_ref, g0h_ref, n_ref, n0h_ref, cr_ref,
              yr_ref, y0h_ref, xt_ref, x0h_ref, acc_ref,
              *, nblk, block, ncls):
    s = pl.program_id(0)
    hi = block - _SHIFTS

    @pl.when(s == 0)
    def _init():
        acc_ref[0] = 0.0
        acc_ref[1] = 0.0
        pn_ref[:, :] = _unit_rows(p_ref[:, :]).astype(jnp.bfloat16)
        ones_ref[:, :] = jnp.ones_like(ones_ref)

    cols = jax.lax.broadcasted_iota(jnp.int16, (block, ncls), 1)
    bzero = jnp.bfloat16(0.0)

    def _rowsum(a16):
        return jax.lax.dot_general(
            a16, ones_ref[:, :], (((1,), (0,)), ((), ())),
            preferred_element_type=jnp.float32)[:, :1]

    @pl.when(jnp.logical_and(s >= 1, s <= nblk))
    def _cross_fix():
        xh16 = jnp.where(s < nblk, xa_ref[:_SHIFTS, :].astype(jnp.bfloat16),
                         x0h_ref[:, :])
        cr_ref[jax.lax.rem(s - 1, 3), hi:, :] = jnp.sum(
            xt_ref[:, :] * xh16, axis=1, keepdims=True, dtype=jnp.float32)

    @pl.when(s < nblk)
    def _pass1():
        x16 = xa_ref[:, :].astype(jnp.bfloat16)
        sq = jnp.sum(x16 * x16, axis=1, keepdims=True, dtype=jnp.float32)
        nb = jnp.sqrt(sq)
        inv = _C / jnp.maximum(nb, 1e-12)
        m = jax.lax.dot_general(
            x16, pn_ref[:, :], (((1,), (1,)), ((), ())),
            preferred_element_type=jnp.float32)
        y16 = (m * inv).astype(jnp.bfloat16)
        yr_ref[jax.lax.rem(s, 3)] = y16
        lse = jnp.log2(jnp.sum(jnp.exp2(m * inv), axis=1, keepdims=True,
                               dtype=jnp.float32))
        lt = _rowsum(jnp.where(cols == ta_ref[0, :, :].astype(jnp.int16),
                               y16, bzero))
        acc_ref[0] += jnp.sum(lse - lt)
        gval = jnp.clip(lt * (1.0 / _C), 0.0, 1.0)
        g_ref[jax.lax.rem(s, 3)] = gval
        n_ref[jax.lax.rem(s, 3)] = nb
        cr_ref[jax.lax.rem(s, 3), :hi, :] = jnp.sum(
            x16[:hi, :] * x16[_SHIFTS:, :], axis=1, keepdims=True,
            dtype=jnp.float32)
        xt_ref[:, :] = x16[hi:, :]

        @pl.when(s == 0)
        def _pin():
            g0h_ref[:, :] = gval[:_SHIFTS, :]
            n0h_ref[:, :] = nb[:_SHIFTS, :]
            y0h_ref[:, :] = y16[:_SHIFTS, :]
            x0h_ref[:, :] = x16[:_SHIFTS, :]

    @pl.when(s >= 2)
    def _pass2():
        in_ring = (s - 1) < nblk
        gb = g_ref[jax.lax.rem(s - 2, 3)]
        gh = jnp.where(in_ring, g_ref[jax.lax.rem(s - 1, 3)][:_SHIFTS, :],
                       g0h_ref[:, :])
        g2 = jnp.concatenate([gb[_SHIFTS:, :], gh], axis=0)
        lam = jnp.clip((gb + 1.0 - g2) * 0.5, 0.0, 1.0)
        oml = 1.0 - lam
        nb = n_ref[jax.lax.rem(s - 2, 3)]
        nh = jnp.where(in_ring, n_ref[jax.lax.rem(s - 1, 3)][:_SHIFTS, :],
                       n0h_ref[:, :])
        ns = jnp.concatenate([nb[_SHIFTS:, :], nh], axis=0)
        cross = cr_ref[jax.lax.rem(s - 2, 3)]
        vn2 = (lam * lam * nb * nb + oml * oml * ns * ns +
               2.0 * (lam * oml) * cross)
        vmax = jnp.maximum(jnp.sqrt(jnp.maximum(vn2, 0.0)), 1e-12)
        alpha = (lam * nb / vmax).astype(jnp.bfloat16)
        beta = (oml * ns / vmax).astype(jnp.bfloat16)
        yb16 = yr_ref[jax.lax.rem(s - 2, 3)]
        yh16 = jnp.where(in_ring, yr_ref[jax.lax.rem(s - 1, 3)][:_SHIFTS, :],
                         y0h_ref[:, :])
        ys16 = jnp.concatenate([yb16[_SHIFTS:, :], yh16], axis=0)
        yv16 = alpha * yb16 + beta * ys16
        lse = jnp.log2(jnp.sum(jnp.exp2(yv16), axis=1, keepdims=True,
                               dtype=jnp.float32))
        l1 = _rowsum(jnp.where(cols == tb_ref[0, :, :].astype(jnp.int16),
                               yv16, bzero))
        l2 = _rowsum(jnp.where(cols == t2b_ref[0, :, :].astype(jnp.int16),
                               yv16, bzero))
        acc_ref[1] += jnp.sum(lse - lam * l1 - oml * l2)

    @pl.when(s == nblk + 1)
    def _fin():
        out_ref[:, :] = jnp.full(
            (1, 1), _LN2 * (acc_ref[0] + acc_ref[1]) / (nblk * block),
            jnp.float32)


@functools.partial(jax.jit, static_argnames=("interpret",))
def kernel(X, T, proxies, interpret=False):
    n, e = X.shape
    ncls = proxies.shape[0]
    block = _BLOCK
    nblk = n // block

    T = T.astype(jnp.int32)
    t_col = T.reshape(nblk, block, 1)
    t2_col = jnp.roll(T, -_SHIFTS).reshape(nblk, block, 1)

    out = pl.pallas_call(
        functools.partial(_nca_body, nblk=nblk, block=block, ncls=ncls),
        grid=(nblk + 2,),
        in_specs=[
            pl.BlockSpec((block, e), lambda s: (jnp.minimum(s, nblk - 1), 0)),
            pl.BlockSpec((ncls, e), lambda s: (0, 0)),
            pl.BlockSpec((1, block, 1),
                         lambda s: (jnp.minimum(s, nblk - 1), 0, 0)),
            pl.BlockSpec((1, block, 1),
                         lambda s: (jnp.maximum(s - 2, 0), 0, 0)),
            pl.BlockSpec((1, block, 1),
                         lambda s: (jnp.maximum(s - 2, 0), 0, 0)),
        ],
        out_specs=pl.BlockSpec((1, 1), lambda s: (0, 0)),
        out_shape=jax.ShapeDtypeStruct((1, 1), jnp.float32),
        scratch_shapes=[
            pltpu.VMEM((ncls, e), jnp.bfloat16),
            pltpu.VMEM((e, 128), jnp.bfloat16),
            pltpu.VMEM((3, block, 1), jnp.float32),
            pltpu.VMEM((_SHIFTS, 1), jnp.float32),
            pltpu.VMEM((3, block, 1), jnp.float32),
            pltpu.VMEM((_SHIFTS, 1), jnp.float32),
            pltpu.VMEM((3, block, 1), jnp.float32),
            pltpu.VMEM((3, block, ncls), jnp.bfloat16),
            pltpu.VMEM((_SHIFTS, ncls), jnp.bfloat16),
            pltpu.VMEM((_SHIFTS, e), jnp.bfloat16),
            pltpu.VMEM((_SHIFTS, e), jnp.bfloat16),
            pltpu.SMEM((2,), jnp.float32),
        ],
        interpret=interpret,
    )(X, proxies, t_col, t_col, t2_col)
    return out[0, 0]
